# Initial kernel scaffold; baseline (speedup 1.0000x reference)
import jax
import jax.numpy as jnp
from jax.experimental import pallas as pl


def kernel(x, W):
    raise NotImplementedError("write your pallas kernel here")


# speedup vs baseline: 1.9839x; 1.9839x over previous
import functools

import jax
import jax.numpy as jnp
from jax import lax
from jax.experimental import pallas as pl
from jax.experimental.pallas import tpu as pltpu
from jax.experimental.pallas import tpu_sc as plsc

D_MODEL = 64
SEQ = 200
HALF = 100
NC, NS = 2, 16
NW = NC * NS
SCALE = 8.0


def _positional_encoding(length, d_model):
    depth = d_model / 2
    pos = jnp.arange(0, length, dtype=jnp.float32)[:, None]
    i = jnp.arange(0, depth, dtype=jnp.float32)
    angle = pos / jnp.power(10000.0, 2.0 * i / depth)
    return jnp.concatenate([jnp.sin(angle), jnp.cos(angle)], axis=-1)


def _sc_embed(x2, W, pe):
    n_chunks = x2.shape[0]
    per_w = n_chunks // NW
    mesh = plsc.VectorSubcoreMesh(core_axis_name="c", subcore_axis_name="s")

    @functools.partial(
        pl.kernel,
        mesh=mesh,
        out_type=jax.ShapeDtypeStruct((n_chunks, HALF, D_MODEL), jnp.float32),
        scratch_types=[
            pltpu.VMEM((HALF,), jnp.int32),
            pltpu.VMEM((HALF, D_MODEL), jnp.float32),
            pltpu.VMEM((SEQ, D_MODEL), jnp.float32),
            pltpu.SemaphoreType.DMA,
        ],
        compiler_params=pltpu.CompilerParams(use_tc_tiling_on_sc=False),
    )
    def k(x_hbm, w_hbm, pe_hbm, out_hbm, idx_v, rows_v, pe_v, sem):
        wid = lax.axis_index("s") * NC + lax.axis_index("c")
        pltpu.sync_copy(pe_hbm, pe_v)
        base = wid * per_w

        def chunk_body(c, carry):
            row = base + c
            pltpu.sync_copy(x_hbm.at[row], idx_v)
            pltpu.async_copy(w_hbm.at[idx_v], rows_v, sem).wait()
            pe_off = lax.rem(row, 2) * HALF

            def row_body(r, carry2):
                l = pe_off + r
                for j in range(D_MODEL // 16):
                    s = pl.ds(j * 16, 16)
                    rows_v[r, s] = rows_v[r, s] * SCALE + pe_v[l, s]
                return carry2

            lax.fori_loop(0, HALF, row_body, 0)
            pltpu.sync_copy(rows_v, out_hbm.at[row])
            return carry

        lax.fori_loop(0, per_w, chunk_body, 0)

    return k(x2, W, pe)


def kernel(x, W):
    B, L = x.shape
    x2 = x.reshape(B * L // HALF, HALF)
    pe = _positional_encoding(L, D_MODEL)
    out = _sc_embed(x2, W, pe)
    return out.reshape(B, L, D_MODEL)


# speedup vs baseline: 3.9045x; 1.9680x over previous
import functools

import jax
import jax.numpy as jnp
from jax import lax
from jax.experimental import pallas as pl
from jax.experimental.pallas import tpu as pltpu
from jax.experimental.pallas import tpu_sc as plsc

D_MODEL = 64
SEQ = 200
HALF = 100
NC, NS = 2, 16
NW = NC * NS
SCALE = 8.0


def _positional_encoding(length, d_model):
    depth = d_model / 2
    pos = jnp.arange(0, length, dtype=jnp.float32)[:, None]
    i = jnp.arange(0, depth, dtype=jnp.float32)
    angle = pos / jnp.power(10000.0, 2.0 * i / depth)
    return jnp.concatenate([jnp.sin(angle), jnp.cos(angle)], axis=-1)


def _sc_embed(x2, W, pe):
    n_chunks = x2.shape[0]
    per_w = n_chunks // NW
    n_iter = per_w // 2
    mesh = plsc.VectorSubcoreMesh(core_axis_name="c", subcore_axis_name="s")

    @functools.partial(
        pl.kernel,
        mesh=mesh,
        out_type=jax.ShapeDtypeStruct((n_chunks, HALF, D_MODEL), jnp.float32),
        scratch_types=[
            pltpu.VMEM((per_w, HALF), jnp.int32),
            pltpu.VMEM((HALF, D_MODEL), jnp.float32),
            pltpu.VMEM((HALF, D_MODEL), jnp.float32),
            pltpu.VMEM((HALF, D_MODEL), jnp.float32),
            pltpu.VMEM((HALF, D_MODEL), jnp.float32),
            pltpu.VMEM((SEQ, D_MODEL), jnp.float32),
            pltpu.SemaphoreType.DMA,
            pltpu.SemaphoreType.DMA,
            pltpu.SemaphoreType.DMA,
            pltpu.SemaphoreType.DMA,
            pltpu.SemaphoreType.DMA,
        ],
        compiler_params=pltpu.CompilerParams(use_tc_tiling_on_sc=False),
    )
    def k(x_hbm, w_hbm, pe_hbm, out_hbm, idx_all, rows0, rows1, comp0,
          comp1, pe_v, isem, gsem0, gsem1, osem0, osem1):
        wid = lax.axis_index("s") * NC + lax.axis_index("c")
        base = wid * per_w
        pltpu.async_copy(x_hbm.at[pl.ds(base, per_w)], idx_all, isem)
        pltpu.sync_copy(pe_hbm, pe_v)
        pltpu.make_async_copy(x_hbm.at[pl.ds(base, per_w)], idx_all, isem).wait()

        rows = (rows0, rows1)
        comp = (comp0, comp1)
        gsem = (gsem0, gsem1)
        osem = (osem0, osem1)

        pltpu.async_copy(w_hbm.at[idx_all.at[0]], rows0, gsem0)
        pltpu.async_copy(w_hbm.at[idx_all.at[1]], rows1, gsem1)

        def iter_body(i, carry):
            for b in range(2):
                c = 2 * i + b
                row = base + c
                pltpu.make_async_copy(
                    w_hbm.at[idx_all.at[c]], rows[b], gsem[b]).wait()

                def _wait_out(bb=b, rr=row):
                    pltpu.make_async_copy(
                        comp[bb], out_hbm.at[rr - 2], osem[bb]).wait()

                pl.when(i > 0)(_wait_out)

                def row_body(r, carry2):
                    l = b * HALF + r
                    for j in range(D_MODEL // 16):
                        s = pl.ds(j * 16, 16)
                        comp[b][r, s] = rows[b][r, s] * SCALE + pe_v[l, s]
                    return carry2

                lax.fori_loop(0, HALF, row_body, 0)
                pltpu.async_copy(comp[b], out_hbm.at[row], osem[b])

                def _next_gather(bb=b, cc=c):
                    pltpu.async_copy(
                        w_hbm.at[idx_all.at[cc + 2]], rows[bb], gsem[bb])

                pl.when(i < n_iter - 1)(_next_gather)
            return carry

        lax.fori_loop(0, n_iter, iter_body, 0)
        pltpu.make_async_copy(comp0, out_hbm.at[base + per_w - 2], osem0).wait()
        pltpu.make_async_copy(comp1, out_hbm.at[base + per_w - 1], osem1).wait()

    return k(x2, W, pe)


def kernel(x, W):
    B, L = x.shape
    x2 = x.reshape(B * L // HALF, HALF)
    pe = _positional_encoding(L, D_MODEL)
    out = _sc_embed(x2, W, pe)
    return out.reshape(B, L, D_MODEL)


# speedup vs baseline: 4.1531x; 1.0637x over previous
import functools

import jax
import jax.numpy as jnp
from jax import lax
from jax.experimental import pallas as pl
from jax.experimental.pallas import tpu as pltpu
from jax.experimental.pallas import tpu_sc as plsc

D_MODEL = 64
HALF = 100
NC, NS = 2, 16
NW = NC * NS
SCALE = 8.0


def _positional_encoding(length, d_model):
    depth = d_model / 2
    pos = jnp.arange(0, length, dtype=jnp.float32)[:, None]
    i = jnp.arange(0, depth, dtype=jnp.float32)
    angle = pos / jnp.power(10000.0, 2.0 * i / depth)
    return jnp.concatenate([jnp.sin(angle), jnp.cos(angle)], axis=-1)


def _sc_embed(x2, W, pe, n_seq, seq):
    n_halves = x2.shape[0]
    per_w = n_halves // NW
    seq_per_w = n_seq // NW
    n_iter = seq_per_w // 2
    mesh = plsc.VectorSubcoreMesh(core_axis_name="c", subcore_axis_name="s")

    @functools.partial(
        pl.kernel,
        mesh=mesh,
        out_type=jax.ShapeDtypeStruct((n_seq, seq, D_MODEL), jnp.float32),
        scratch_types=[
            pltpu.VMEM((per_w, HALF), jnp.int32),
            pltpu.VMEM((seq, D_MODEL), jnp.float32),
            pltpu.VMEM((seq, D_MODEL), jnp.float32),
            pltpu.VMEM((seq, D_MODEL), jnp.float32),
            pltpu.VMEM((seq, D_MODEL), jnp.float32),
            pltpu.VMEM((seq, D_MODEL), jnp.float32),
            pltpu.SemaphoreType.DMA,
            pltpu.SemaphoreType.DMA,
            pltpu.SemaphoreType.DMA,
            pltpu.SemaphoreType.DMA,
            pltpu.SemaphoreType.DMA,
        ],
        compiler_params=pltpu.CompilerParams(use_tc_tiling_on_sc=False),
    )
    def k(x_hbm, w_hbm, pe_hbm, out_hbm, idx_all, rows0, rows1, comp0,
          comp1, pe_v, isem, gsem0, gsem1, osem0, osem1):
        wid = lax.axis_index("s") * NC + lax.axis_index("c")
        hbase = wid * per_w
        sbase = wid * seq_per_w
        pltpu.async_copy(x_hbm.at[pl.ds(hbase, per_w)], idx_all, isem)
        pltpu.sync_copy(pe_hbm, pe_v)
        pltpu.make_async_copy(x_hbm.at[pl.ds(hbase, per_w)], idx_all, isem).wait()

        rows = (rows0, rows1)
        comp = (comp0, comp1)
        gsem = (gsem0, gsem1)
        osem = (osem0, osem1)

        def start_gathers(c, b):
            pltpu.async_copy(
                w_hbm.at[idx_all.at[2 * c]], rows[b].at[pl.ds(0, HALF)],
                gsem[b])
            pltpu.async_copy(
                w_hbm.at[idx_all.at[2 * c + 1]], rows[b].at[pl.ds(HALF, HALF)],
                gsem[b])

        def wait_gathers(c, b):
            pltpu.make_async_copy(
                w_hbm.at[idx_all.at[2 * c]], rows[b].at[pl.ds(0, HALF)],
                gsem[b]).wait()
            pltpu.make_async_copy(
                w_hbm.at[idx_all.at[2 * c + 1]], rows[b].at[pl.ds(HALF, HALF)],
                gsem[b]).wait()

        start_gathers(0, 0)
        start_gathers(1, 1)

        def iter_body(i, carry):
            for b in range(2):
                c = 2 * i + b
                s = sbase + c
                wait_gathers(c, b)

                def _wait_out(bb=b, ss=s):
                    pltpu.make_async_copy(
                        comp[bb], out_hbm.at[ss - 2], osem[bb]).wait()

                pl.when(i > 0)(_wait_out)

                def row_body(r, carry2):
                    for j in range(D_MODEL // 16):
                        sl = pl.ds(j * 16, 16)
                        comp[b][r, sl] = rows[b][r, sl] * SCALE + pe_v[r, sl]
                    return carry2

                lax.fori_loop(0, seq, row_body, 0)
                pltpu.async_copy(comp[b], out_hbm.at[s], osem[b])

                def _next_gather(bb=b, cc=c):
                    start_gathers(cc + 2, bb)

                pl.when(i < n_iter - 1)(_next_gather)
            return carry

        lax.fori_loop(0, n_iter, iter_body, 0)
        pltpu.make_async_copy(comp0, out_hbm.at[sbase + seq_per_w - 2], osem0).wait()
        pltpu.make_async_copy(comp1, out_hbm.at[sbase + seq_per_w - 1], osem1).wait()

    return k(x2, W, pe)


def kernel(x, W):
    B, L = x.shape
    x2 = x.reshape(B * L // HALF, HALF)
    pe = _positional_encoding(L, D_MODEL)
    return _sc_embed(x2, W, pe, B, L)
